# Initial kernel scaffold; baseline (speedup 1.0000x reference)
import jax
import jax.numpy as jnp
from jax.experimental import pallas as pl


def kernel(x, embeddings):
    raise NotImplementedError("write your pallas kernel here")


# speedup vs baseline: 1.4427x; 1.4427x over previous
import functools

import jax
import jax.numpy as jnp
from jax import lax
from jax.experimental import pallas as pl
from jax.experimental.pallas import tpu as pltpu
from jax.experimental.pallas import tpu_sc as plsc

_NUM_EMB = 1024
_DIM = 64
_ROWS = 16384
_BETA = 0.25
_BLK = 1024


def _argmin_body(x_ref, emb_ref, idx_ref, loss_ref, emb_t_ref, acc_ref):
    i = pl.program_id(0)
    x = x_ref[...]
    emb = emb_ref[...]
    sim = jnp.dot(x, emb, preferred_element_type=jnp.float32)
    rn = jnp.sum(x * x, axis=1, keepdims=True)
    en = jnp.sum(emb * emb, axis=0)
    d = rn + en[None, :] - 2.0 * sim
    m = jnp.min(d, axis=1, keepdims=True)
    ii = lax.broadcasted_iota(jnp.int32, d.shape, 1)
    idx = jnp.min(jnp.where(d == m, ii, jnp.int32(2**30)), axis=1)
    idx_ref[...] = idx

    @pl.when(i == 0)
    def _():
        acc_ref[0] = 0.0
        emb_t_ref[...] = emb.T

    acc_ref[0] += jnp.sum(m)

    @pl.when(i == pl.num_programs(0) - 1)
    def _():
        c = acc_ref[0] / jnp.float32(_ROWS * _DIM)
        loss_ref[...] = jnp.full((1, 1), _BETA * c + c, jnp.float32)


_argmin_call = pl.pallas_call(
    _argmin_body,
    grid=(_ROWS // _BLK,),
    in_specs=[
        pl.BlockSpec((_BLK, _DIM), lambda i: (i, 0)),
        pl.BlockSpec((_DIM, _NUM_EMB), lambda i: (0, 0)),
    ],
    out_specs=[
        pl.BlockSpec((_BLK,), lambda i: (i,)),
        pl.BlockSpec((1, 1), lambda i: (0, 0)),
        pl.BlockSpec((_NUM_EMB, _DIM), lambda i: (0, 0)),
    ],
    out_shape=[
        jax.ShapeDtypeStruct((_ROWS,), jnp.int32),
        jax.ShapeDtypeStruct((1, 1), jnp.float32),
        jax.ShapeDtypeStruct((_NUM_EMB, _DIM), jnp.float32),
    ],
    scratch_shapes=[pltpu.SMEM((1,), jnp.float32)],
    compiler_params=pltpu.CompilerParams(
        dimension_semantics=("arbitrary",),
    ),
)

_NC = 2
_NS = 16
_NW = _NC * _NS
_BPW = _ROWS // _NW
_CH = 128


@functools.cache
def _sc_gather_fn():
    mesh = plsc.VectorSubcoreMesh(
        core_axis_name="c", subcore_axis_name="s",
        num_cores=_NC, num_subcores=_NS,
    )

    @functools.partial(
        pl.kernel,
        out_type=jax.ShapeDtypeStruct((_ROWS, _DIM), jnp.float32),
        mesh=mesh,
        scratch_types=[
            pltpu.VMEM((_BPW,), jnp.int32),
            pltpu.VMEM((_BPW, _DIM), jnp.float32),
            pltpu.SemaphoreType.DMA,
        ],
        compiler_params=pltpu.CompilerParams(use_tc_tiling_on_sc=False),
    )
    def _sc_gather(table_hbm, idx_hbm, out_hbm, idx_v, rows_v, sem):
        wid = lax.axis_index("s") * _NC + lax.axis_index("c")
        base = wid * _BPW
        pltpu.sync_copy(idx_hbm.at[pl.ds(base, _BPW)], idx_v)
        copies = [
            pltpu.async_copy(
                table_hbm.at[idx_v.at[pl.ds(k * _CH, _CH)]],
                rows_v.at[pl.ds(k * _CH, _CH)],
                sem,
            )
            for k in range(_BPW // _CH)
        ]
        for c in copies:
            c.wait()
        pltpu.sync_copy(rows_v, out_hbm.at[pl.ds(base, _BPW)])

    return _sc_gather


def kernel(x, embeddings):
    x2 = x.reshape(_ROWS, _DIM)
    idx, loss, table = _argmin_call(x2, embeddings)
    q = _sc_gather_fn()(table, idx)
    return q.reshape(x.shape), loss[0, 0]


# speedup vs baseline: 1.9587x; 1.3576x over previous
import functools

import jax
import jax.numpy as jnp
from jax import lax
from jax.experimental import pallas as pl
from jax.experimental.pallas import tpu as pltpu
from jax.experimental.pallas import tpu_sc as plsc

_NUM_EMB = 1024
_DIM = 64
_ROWS = 16384
_BETA = 0.25
_BLK = 2048


def _argmin_body(x_ref, emb_ref, idx_ref, loss_ref, emb_t_ref, acc_ref):
    i = pl.program_id(0)
    x = x_ref[...]
    emb = emb_ref[...]
    sim = jnp.dot(x, emb, preferred_element_type=jnp.float32)
    rn = jnp.sum(x * x, axis=1, keepdims=True)
    en = jnp.sum(emb * emb, axis=0)
    d = rn + en[None, :] - 2.0 * sim
    m = jnp.min(d, axis=1, keepdims=True)
    ii = lax.broadcasted_iota(jnp.int32, d.shape, 1).astype(jnp.float32)
    idx_f = jnp.min(jnp.where(d == m, ii, jnp.float32(2048.0)), axis=1)
    idx_ref[...] = idx_f.astype(jnp.int32).reshape(idx_ref.shape)

    @pl.when(i == 0)
    def _():
        acc_ref[0] = 0.0
        emb_t_ref[...] = emb.T

    acc_ref[0] += jnp.sum(m)

    @pl.when(i == pl.num_programs(0) - 1)
    def _():
        c = acc_ref[0] / jnp.float32(_ROWS * _DIM)
        loss_ref[...] = jnp.full((1, 1), _BETA * c + c, jnp.float32)


_argmin_call = pl.pallas_call(
    _argmin_body,
    grid=(_ROWS // _BLK,),
    in_specs=[
        pl.BlockSpec((_BLK, _DIM), lambda i: (i, 0)),
        pl.BlockSpec((_DIM, _NUM_EMB), lambda i: (0, 0)),
    ],
    out_specs=[
        pl.BlockSpec((_BLK // 128, 128), lambda i: (i, 0)),
        pl.BlockSpec((1, 1), lambda i: (0, 0)),
        pl.BlockSpec((_NUM_EMB, _DIM), lambda i: (0, 0)),
    ],
    out_shape=[
        jax.ShapeDtypeStruct((_ROWS // 128, 128), jnp.int32),
        jax.ShapeDtypeStruct((1, 1), jnp.float32),
        jax.ShapeDtypeStruct((_NUM_EMB, _DIM), jnp.float32),
    ],
    scratch_shapes=[pltpu.SMEM((1,), jnp.float32)],
    compiler_params=pltpu.CompilerParams(
        dimension_semantics=("arbitrary",),
    ),
)

_NC = 2
_NS = 16
_NW = _NC * _NS
_BPW = _ROWS // _NW
_CH = 128


@functools.cache
def _sc_gather_fn():
    mesh = plsc.VectorSubcoreMesh(
        core_axis_name="c", subcore_axis_name="s",
        num_cores=_NC, num_subcores=_NS,
    )

    @functools.partial(
        pl.kernel,
        out_type=jax.ShapeDtypeStruct((_ROWS, _DIM), jnp.float32),
        mesh=mesh,
        scratch_types=[
            pltpu.VMEM((_BPW,), jnp.int32),
            pltpu.VMEM((_BPW, _DIM), jnp.float32),
            pltpu.SemaphoreType.DMA,
        ],
        compiler_params=pltpu.CompilerParams(use_tc_tiling_on_sc=False),
    )
    def _sc_gather(table_hbm, idx_hbm, out_hbm, idx_v, rows_v, sem):
        wid = lax.axis_index("s") * _NC + lax.axis_index("c")
        base = wid * _BPW
        pltpu.sync_copy(idx_hbm.at[pl.ds(base, _BPW)], idx_v)
        copies = [
            pltpu.async_copy(
                table_hbm.at[idx_v.at[pl.ds(k * _CH, _CH)]],
                rows_v.at[pl.ds(k * _CH, _CH)],
                sem,
            )
            for k in range(_BPW // _CH)
        ]
        for c in copies:
            c.wait()
        pltpu.sync_copy(rows_v, out_hbm.at[pl.ds(base, _BPW)])

    return _sc_gather


def kernel(x, embeddings):
    x2 = x.reshape(_ROWS, _DIM)
    idx, loss, table = _argmin_call(x2, embeddings)
    q = _sc_gather_fn()(table, idx.reshape(_ROWS))
    return q.reshape(x.shape), loss[0, 0]


# speedup vs baseline: 2.0384x; 1.0407x over previous
import functools

import jax
import jax.numpy as jnp
from jax import lax
from jax.experimental import pallas as pl
from jax.experimental.pallas import tpu as pltpu
from jax.experimental.pallas import tpu_sc as plsc

_NUM_EMB = 1024
_DIM = 64
_ROWS = 16384
_BETA = 0.25
_BLK = 2048


def _argmin_body(x_ref, emb_ref, idx_ref, loss_ref, acc_ref):
    i = pl.program_id(0)
    x = x_ref[...]
    emb = emb_ref[...]
    sim = jnp.dot(x, emb, preferred_element_type=jnp.float32)
    rn = jnp.sum(x * x, axis=1, keepdims=True)
    en = jnp.sum(emb * emb, axis=0)
    d = rn + en[None, :] - 2.0 * sim
    m = jnp.min(d, axis=1, keepdims=True)
    ii = lax.broadcasted_iota(jnp.int32, d.shape, 1).astype(jnp.float32)
    idx_f = jnp.min(jnp.where(d == m, ii, jnp.float32(2048.0)), axis=1)
    idx_ref[...] = idx_f.astype(jnp.int32).reshape(idx_ref.shape)

    @pl.when(i == 0)
    def _():
        acc_ref[0] = 0.0

    acc_ref[0] += jnp.sum(m)

    @pl.when(i == pl.num_programs(0) - 1)
    def _():
        c = acc_ref[0] / jnp.float32(_ROWS * _DIM)
        loss_ref[...] = jnp.full((1, 1), _BETA * c + c, jnp.float32)


_argmin_call = pl.pallas_call(
    _argmin_body,
    grid=(_ROWS // _BLK,),
    in_specs=[
        pl.BlockSpec((_BLK, _DIM), lambda i: (i, 0)),
        pl.BlockSpec((_DIM, _NUM_EMB), lambda i: (0, 0)),
    ],
    out_specs=[
        pl.BlockSpec((_BLK // 128, 128), lambda i: (i, 0)),
        pl.BlockSpec((1, 1), lambda i: (0, 0)),
    ],
    out_shape=[
        jax.ShapeDtypeStruct((_ROWS // 128, 128), jnp.int32),
        jax.ShapeDtypeStruct((1, 1), jnp.float32),
    ],
    scratch_shapes=[pltpu.SMEM((1,), jnp.float32)],
    compiler_params=pltpu.CompilerParams(
        dimension_semantics=("arbitrary",),
    ),
)

_NC = 2
_NS = 16
_NW = _NC * _NS
_BPW = _ROWS // _NW
_CH = 128


@functools.cache
def _sc_gather_fn():
    mesh = plsc.VectorSubcoreMesh(
        core_axis_name="c", subcore_axis_name="s",
        num_cores=_NC, num_subcores=_NS,
    )

    @functools.partial(
        pl.kernel,
        out_type=jax.ShapeDtypeStruct((_ROWS, _DIM), jnp.float32),
        mesh=mesh,
        scratch_types=[
            pltpu.VMEM((_BPW,), jnp.int32),
            pltpu.VMEM((_BPW, _DIM), jnp.float32),
            pltpu.SemaphoreType.DMA,
        ],
        compiler_params=pltpu.CompilerParams(use_tc_tiling_on_sc=False),
    )
    def _sc_gather(table_hbm, idx_hbm, out_hbm, idx_v, rows_v, sem):
        wid = lax.axis_index("s") * _NC + lax.axis_index("c")
        base = wid * _BPW
        pltpu.sync_copy(idx_hbm.at[pl.ds(base, _BPW)], idx_v)
        copies = [
            pltpu.async_copy(
                table_hbm.at[idx_v.at[pl.ds(k * _CH, _CH)]],
                rows_v.at[pl.ds(k * _CH, _CH)],
                sem,
            )
            for k in range(_BPW // _CH)
        ]
        for c in copies:
            c.wait()
        pltpu.sync_copy(rows_v, out_hbm.at[pl.ds(base, _BPW)])

    return _sc_gather


def kernel(x, embeddings):
    x2 = x.reshape(_ROWS, _DIM)
    idx, loss = _argmin_call(x2, embeddings)
    table = embeddings.T
    q = _sc_gather_fn()(table, idx.reshape(_ROWS))
    return q.reshape(x.shape), loss[0, 0]
